# baseline (speedup 1.0000x reference)
import functools

import jax
import jax.numpy as jnp
from jax.experimental import pallas as pl
from jax.experimental.pallas import tpu as pltpu

_IN = 1024
_OUT = 2048
_MAXL = 8
_T = 256
_NTOK = 8192
_NB = _NTOK // _T + _MAXL
_DEPTH = (3, 3, 4, 4, 5, 5, 5, 5)
_MAXD = 5
_HALF = _OUT // 2

_SLOT = []
_slot_base = 0
for _e in range(_MAXL):
    _SLOT.append([_slot_base + _j for _j in range(_DEPTH[_e] - 1)])
    _slot_base += _DEPTH[_e] - 1
_NSLOTS = _slot_base

_RSEL_ROWS = []
for _e in range(_MAXL):
    _s = _SLOT[_e]
    _row = [_s[0], _s[0]] + [_s[min(_j, len(_s) - 1)] for _j in range(1, _MAXD - 1)]
    _RSEL_ROWS.append(_row)
_LAST_SLOT = [_SLOT[_e][-1] for _e in range(_MAXL)]


def _mlp_body(sel_ref, rsel_ref, islayer_ref, islast_ref,
              x_ref, w0_ref, b0_ref, wr_ref, brc_ref,
              o_ref, h_ref, acc_ref):
    b = pl.program_id(0)
    l = pl.program_id(1)
    k = pl.program_id(2)
    do = islayer_ref[b, l] == 1
    last = islast_ref[b, l] == 1

    @pl.when(do & (l == 0) & (k == 0))
    def _first():
        acc = jax.lax.dot_general(
            w0_ref[0], x_ref[...], (((1,), (1,)), ((), ())),
            preferred_element_type=jnp.float32) + b0_ref[0]
        h_ref[...] = jnp.maximum(acc, 0.0)

    @pl.when(do & (l > 0) & (k == 0))
    def _mid0():
        acc_ref[...] = jax.lax.dot_general(
            wr_ref[0], h_ref[0:_HALF, :], (((1,), (0,)), ((), ())),
            preferred_element_type=jnp.float32)

    @pl.when(do & (l > 0) & (k == 1))
    def _mid1():
        acc = acc_ref[...] + jax.lax.dot_general(
            wr_ref[0], h_ref[_HALF:, :], (((1,), (0,)), ((), ())),
            preferred_element_type=jnp.float32) + brc_ref[0]

        @pl.when(jnp.logical_not(last))
        def _():
            h_ref[...] = jnp.maximum(acc, 0.0)

        @pl.when(last)
        def _():
            o_ref[...] = acc.T


def _expert_mlp(x_sorted, sel, rsel, is_layer, is_last,
                w0_bank, b0_bank, wr_bank, brc_bank):
    grid_spec = pltpu.PrefetchScalarGridSpec(
        num_scalar_prefetch=4,
        grid=(_NB, _MAXD, 2),
        in_specs=[
            pl.BlockSpec((_T, _IN), lambda b, l, k, *p: (b, 0)),
            pl.BlockSpec((1, _OUT, _IN), lambda b, l, k, sel, rsel, *p: (sel[b], 0, 0)),
            pl.BlockSpec((1, _OUT, 1), lambda b, l, k, sel, rsel, *p: (sel[b], 0, 0)),
            pl.BlockSpec((1, _OUT, _HALF),
                         lambda b, l, k, sel, rsel, *p: (rsel[b, l], 0, k)),
            pl.BlockSpec((1, _OUT, 1), lambda b, l, k, sel, rsel, *p: (rsel[b, l], 0, 0)),
        ],
        out_specs=pl.BlockSpec((_T, _OUT), lambda b, l, k, *p: (b, 0)),
        scratch_shapes=[pltpu.VMEM((_OUT, _T), jnp.float32),
                        pltpu.VMEM((_OUT, _T), jnp.float32)],
    )
    return pl.pallas_call(
        _mlp_body,
        grid_spec=grid_spec,
        out_shape=jax.ShapeDtypeStruct((_NB * _T, _OUT), jnp.float32),
        compiler_params=pltpu.CompilerParams(
            dimension_semantics=("arbitrary", "arbitrary", "arbitrary")),
    )(sel, rsel, is_layer, is_last,
      x_sorted, w0_bank, b0_bank, wr_bank, brc_bank)


def _pack_weights(params):
    w0s, b0s, wrs, brs = [], [], [], []
    for e in range(_MAXL):
        layers = params[e]
        W0, B0 = layers[0]
        d0 = W0.shape[0]
        w0s.append(jnp.pad(W0, ((0, _OUT - d0), (0, 0))))
        b0s.append(jnp.pad(B0, (0, _OUT - d0)))
        for j in range(1, _DEPTH[e]):
            W, B = layers[j]
            dout, din = W.shape
            wrs.append(jnp.pad(W, ((0, _OUT - dout), (0, _OUT - din))))
            brs.append(jnp.pad(B, (0, _OUT - dout)))
    return (jnp.stack(w0s), jnp.stack(b0s)[:, :, None],
            jnp.stack(wrs), jnp.stack(brs)[:, :, None])


def kernel(x, path_lengths, params):
    b, n, d = x.shape
    xf = x.reshape(b * n, d)
    plf = jnp.clip(path_lengths.reshape(b * n), 0, _MAXL - 1)

    onehot = (plf[:, None] == jnp.arange(_MAXL, dtype=jnp.int32)[None, :])
    oh32 = onehot.astype(jnp.int32)
    counts = jnp.sum(oh32, axis=0)
    padded = ((counts + _T - 1) // _T) * _T
    ends = jnp.cumsum(padded)
    starts = ends - padded
    ranks_all = jnp.cumsum(oh32, axis=0) - oh32
    rank = jnp.take_along_axis(ranks_all, plf[:, None], axis=1)[:, 0]
    dest = starts[plf] + rank

    used_blocks = ends[-1] // _T
    bid = jnp.arange(_NB, dtype=jnp.int32)
    src_blk = jnp.minimum(bid, used_blocks - 1)
    blk_expert = jnp.searchsorted(ends, src_blk * _T, side="right").astype(jnp.int32)
    sel = blk_expert

    used = (bid < used_blocks)
    depth_b = jnp.array(_DEPTH, dtype=jnp.int32)[sel]
    lvec = jnp.arange(_MAXD, dtype=jnp.int32)
    is_layer = (used[:, None] & (lvec[None, :] < depth_b[:, None])).astype(jnp.int32)
    is_last = (used[:, None] & (lvec[None, :] == depth_b[:, None] - 1)).astype(jnp.int32)
    rsel_tab = jnp.array(_RSEL_ROWS, dtype=jnp.int32)
    last_tab = jnp.array(_LAST_SLOT, dtype=jnp.int32)
    rsel = jnp.where(used[:, None], rsel_tab[sel], last_tab[sel][:, None])

    x_sorted = jnp.zeros((_NB * _T, _IN), dtype=jnp.float32).at[dest].set(xf)

    banks = _pack_weights(params)
    y_sorted = _expert_mlp(x_sorted, sel, rsel, is_layer, is_last, *banks)

    out = y_sorted[dest]
    return out.reshape(b, n, _OUT)


# speedup vs baseline: 1.1692x; 1.1692x over previous
import jax
import jax.numpy as jnp
from jax.experimental import pallas as pl
from jax.experimental.pallas import tpu as pltpu

_IN = 1024
_OUT = 2048
_MAXL = 8
_T = 256
_G = 4
_S = _T * _G
_NTOK = 8192
_NSUP = _NTOK // _S + _MAXL
_NBF = _NSUP * _G
_NPH = 6
_DEPTH = (3, 3, 4, 4, 5, 5, 5, 5)

_SLOT = []
_sb = 0
for _e in range(_MAXL):
    _SLOT.append([_sb + _j for _j in range(_DEPTH[_e])])
    _sb += _DEPTH[_e]
_NSLOTS = _sb

_PS_ROWS = []
for _e in range(_MAXL):
    _D = _DEPTH[_e]
    _row = [(_SLOT[_e][_l] if _l <= _D - 2 else -1) for _l in range(_NPH - 1)]
    _row.append(_SLOT[_e][_D - 1])
    _PS_ROWS.append(_row)


def _mlp_body(pslot_ref, dohid_ref, dofin_ref, fineu_ref,
              x_ref, w_ref, bc_ref, br_ref, o_ref, h_ref):
    sb = pl.program_id(0)
    l = pl.program_id(1)
    g = pl.program_id(2)
    fu = fineu_ref[sb, g] == 1
    hid = (dohid_ref[sb, l] == 1) & fu
    fin = (dofin_ref[sb, l] == 1) & fu

    @pl.when(hid & (l == 0))
    def _first():
        acc = jax.lax.dot_general(
            w_ref[0][:, 0:_IN], x_ref[...], (((1,), (1,)), ((), ())),
            preferred_element_type=jnp.float32) + bc_ref[0]
        h_ref[g] = jnp.maximum(acc, 0.0)

    @pl.when(hid & (l > 0))
    def _mid():
        acc = jax.lax.dot_general(
            w_ref[0], h_ref[g], (((1,), (0,)), ((), ())),
            preferred_element_type=jnp.float32) + bc_ref[0]
        h_ref[g] = jnp.maximum(acc, 0.0)

    @pl.when(fin)
    def _final():
        o_ref[...] = jax.lax.dot_general(
            h_ref[g], w_ref[0], (((0,), (1,)), ((), ())),
            preferred_element_type=jnp.float32) + br_ref[0]


def _expert_mlp(x_sorted, pslot, dohid, dofin, fineu, w_bank, bc_bank, br_bank):
    grid_spec = pltpu.PrefetchScalarGridSpec(
        num_scalar_prefetch=4,
        grid=(_NSUP, _NPH, _G),
        in_specs=[
            pl.BlockSpec(
                (_T, _IN),
                lambda sb, l, g, ps, *p: (sb * _G + jnp.where(l == 0, g, _G - 1), 0)),
            pl.BlockSpec(
                (1, _OUT, _OUT),
                lambda sb, l, g, ps, *p: (ps[sb, l], 0, 0)),
            pl.BlockSpec(
                (1, _OUT, 1),
                lambda sb, l, g, ps, *p: (ps[sb, l], 0, 0)),
            pl.BlockSpec(
                (1, 1, _OUT),
                lambda sb, l, g, ps, *p: (ps[sb, l], 0, 0)),
        ],
        out_specs=pl.BlockSpec(
            (_T, _OUT),
            lambda sb, l, g, ps, *p: (sb * _G + jnp.where(l == _NPH - 1, g, 0), 0)),
        scratch_shapes=[pltpu.VMEM((_G, _OUT, _T), jnp.float32)],
    )
    return pl.pallas_call(
        _mlp_body,
        grid_spec=grid_spec,
        out_shape=jax.ShapeDtypeStruct((_NBF * _T, _OUT), jnp.float32),
        compiler_params=pltpu.CompilerParams(
            dimension_semantics=("arbitrary", "arbitrary", "arbitrary"),
            fuse_transposed_lhs_in_matmul=True),
    )(pslot, dohid, dofin, fineu,
      x_sorted, w_bank, bc_bank, br_bank)


def _pack_weights(params):
    ws, bs = [], []
    for e in range(_MAXL):
        for j in range(_DEPTH[e]):
            W, B = params[e][j]
            dout, din = W.shape
            ws.append(jnp.pad(W, ((0, _OUT - dout), (0, _OUT - din))))
            bs.append(jnp.pad(B, (0, _OUT - dout)))
    b = jnp.stack(bs)
    return jnp.stack(ws), b[:, :, None], b[:, None, :]


def kernel(x, path_lengths, params):
    b, n, d = x.shape
    xf = x.reshape(b * n, d)
    plf = jnp.clip(path_lengths.reshape(b * n), 0, _MAXL - 1)

    onehot = (plf[:, None] == jnp.arange(_MAXL, dtype=jnp.int32)[None, :])
    oh32 = onehot.astype(jnp.int32)
    counts = jnp.sum(oh32, axis=0)
    padded = ((counts + _S - 1) // _S) * _S
    ends = jnp.cumsum(padded)
    starts = ends - padded
    ranks_all = jnp.cumsum(oh32, axis=0) - oh32
    rank = jnp.take_along_axis(ranks_all, plf[:, None], axis=1)[:, 0]
    dest = starts[plf] + rank

    used_supers = ends[-1] // _S
    sbid = jnp.arange(_NSUP, dtype=jnp.int32)
    src_sb = jnp.minimum(sbid, used_supers - 1)
    sel = jnp.searchsorted(ends, src_sb * _S, side="right").astype(jnp.int32)
    sup_used = (sbid < used_supers)

    fbid = jnp.arange(_NBF, dtype=jnp.int32)
    fsel = sel[jnp.minimum(fbid // _G, used_supers - 1)]
    fineu = ((fbid // _G < used_supers)
             & (fbid * _T < starts[fsel] + counts[fsel])).astype(jnp.int32)
    fineu = fineu.reshape(_NSUP, _G)

    ps_tab = jnp.array(_PS_ROWS, dtype=jnp.int32)
    pslot = jnp.where(sup_used[:, None], ps_tab[sel], -1).reshape(-1)
    pidx = jnp.arange(pslot.shape[0], dtype=jnp.int32)
    lastvalid = jax.lax.cummax(jnp.where(pslot >= 0, pidx, -1))
    pslot = pslot[jnp.maximum(lastvalid, 0)].reshape(_NSUP, _NPH)

    depth_b = jnp.array(_DEPTH, dtype=jnp.int32)[sel]
    lvec = jnp.arange(_NPH, dtype=jnp.int32)
    dohid = (sup_used[:, None] & (lvec[None, :] <= depth_b[:, None] - 2)).astype(jnp.int32)
    dofin = (sup_used[:, None] & (lvec[None, :] == _NPH - 1)).astype(jnp.int32)

    x_sorted = jnp.zeros((_NBF * _T, _IN), dtype=jnp.float32).at[dest].set(xf)

    w_bank, bc_bank, br_bank = _pack_weights(params)
    y_sorted = _expert_mlp(x_sorted, pslot, dohid, dofin, fineu,
                           w_bank, bc_bank, br_bank)

    out = y_sorted[dest]
    return out.reshape(b, n, _OUT)


# speedup vs baseline: 1.7890x; 1.5301x over previous
import jax
import jax.numpy as jnp
from jax.experimental import pallas as pl
from jax.experimental.pallas import tpu as pltpu

_IN = 1024
_OUT = 2048
_MAXL = 8
_T = 256
_G = 4
_S = _T * _G
_NTOK = 8192
_NSUP = _NTOK // _S + _MAXL
_NBF = _NSUP * _G
_NPH = 6
_NP = _NSUP * _NPH
_DEPTH = (3, 3, 4, 4, 5, 5, 5, 5)

_SLOT = []
_sb = 0
for _e in range(_MAXL):
    _SLOT.append([_sb + _j for _j in range(_DEPTH[_e])])
    _sb += _DEPTH[_e]
_NSLOTS = _sb

_DINS = []
for _e in range(_MAXL):
    for _j in range(_DEPTH[_e]):
        _DINS.append(_IN if _j == 0 else _OUT)

_PS_ROWS = []
for _e in range(_MAXL):
    _D = _DEPTH[_e]
    _row = [(_SLOT[_e][_l] if _l <= _D - 2 else -1) for _l in range(_NPH - 1)]
    _row.append(_SLOT[_e][_D - 1])
    _PS_ROWS.append(_row)


def _switch_dma(slot, w_refs, dst_ref, sem_ref, buf, start):
    for i in range(_NSLOTS):
        @pl.when(slot == i)
        def _(i=i):
            cp = pltpu.make_async_copy(
                w_refs[i], dst_ref.at[buf, :, pl.ds(0, _DINS[i])], sem_ref.at[buf])
            if start:
                cp.start()
            else:
                cp.wait()


def _mlp_body(psff_ref, dohid_ref, dofin_ref, fineu_ref,
              wslot_ref, curbuf_ref, islot_ref, ibuf_ref,
              x_ref, bc_ref, br_ref, *rest):
    w_refs = rest[:_NSLOTS]
    o_ref = rest[_NSLOTS]
    wbuf_ref, h_ref, sem_ref = rest[_NSLOTS + 1:]
    sb = pl.program_id(0)
    l = pl.program_id(1)
    g = pl.program_id(2)
    p = sb * _NPH + l

    @pl.when(g == 0)
    def _dma_mgmt():
        @pl.when(p == 0)
        def _bootstrap():
            _switch_dma(wslot_ref[0], w_refs, wbuf_ref, sem_ref,
                        curbuf_ref[0], start=True)

        islot = islot_ref[p]

        @pl.when(islot >= 0)
        def _issue_next():
            _switch_dma(islot, w_refs, wbuf_ref, sem_ref,
                        ibuf_ref[p], start=True)

        wslot = wslot_ref[p]

        @pl.when(wslot >= 0)
        def _wait_cur():
            _switch_dma(wslot, w_refs, wbuf_ref, sem_ref,
                        curbuf_ref[p], start=False)

    fu = fineu_ref[sb, g] == 1
    hid = (dohid_ref[sb, l] == 1) & fu
    fin = (dofin_ref[sb, l] == 1) & fu
    cur = curbuf_ref[p]

    @pl.when(hid & (l == 0))
    def _first():
        acc = jax.lax.dot_general(
            wbuf_ref[cur, :, 0:_IN], x_ref[...], (((1,), (1,)), ((), ())),
            preferred_element_type=jnp.float32) + bc_ref[0]
        h_ref[g] = jnp.maximum(acc, 0.0)

    @pl.when(hid & (l > 0))
    def _mid():
        acc = jax.lax.dot_general(
            wbuf_ref[cur], h_ref[g], (((1,), (0,)), ((), ())),
            preferred_element_type=jnp.float32) + bc_ref[0]
        h_ref[g] = jnp.maximum(acc, 0.0)

    @pl.when(fin)
    def _final():
        o_ref[...] = jax.lax.dot_general(
            h_ref[g], wbuf_ref[cur], (((0,), (1,)), ((), ())),
            preferred_element_type=jnp.float32) + br_ref[0]


def _expert_mlp(x_sorted, psff, dohid, dofin, fineu,
                wslot, curbuf, islot, ibuf, weights, bc_bank, br_bank):
    grid_spec = pltpu.PrefetchScalarGridSpec(
        num_scalar_prefetch=8,
        grid=(_NSUP, _NPH, _G),
        in_specs=[
            pl.BlockSpec(
                (_T, _IN),
                lambda sb, l, g, ps, *p: (sb * _G + jnp.where(l == 0, g, _G - 1), 0)),
            pl.BlockSpec(
                (1, _OUT, 1),
                lambda sb, l, g, ps, *p: (ps[sb, l], 0, 0)),
            pl.BlockSpec(
                (1, 1, _OUT),
                lambda sb, l, g, ps, *p: (ps[sb, l], 0, 0)),
        ] + [pl.BlockSpec(memory_space=pltpu.MemorySpace.HBM)] * _NSLOTS,
        out_specs=pl.BlockSpec(
            (_T, _OUT),
            lambda sb, l, g, ps, *p: (sb * _G + jnp.where(l == _NPH - 1, g, 0), 0)),
        scratch_shapes=[
            pltpu.VMEM((2, _OUT, _OUT), jnp.float32),
            pltpu.VMEM((_G, _OUT, _T), jnp.float32),
            pltpu.SemaphoreType.DMA((2,)),
        ],
    )
    return pl.pallas_call(
        _mlp_body,
        grid_spec=grid_spec,
        out_shape=jax.ShapeDtypeStruct((_NBF * _T, _OUT), jnp.float32),
        compiler_params=pltpu.CompilerParams(
            dimension_semantics=("arbitrary", "arbitrary", "arbitrary"),
            fuse_transposed_lhs_in_matmul=True),
    )(psff, dohid, dofin, fineu, wslot, curbuf, islot, ibuf,
      x_sorted, bc_bank, br_bank, *weights)


def _prep_weights(params):
    ws, bs = [], []
    for e in range(_MAXL):
        for j in range(_DEPTH[e]):
            W, B = params[e][j]
            dout, din = W.shape
            if j == 0:
                W = jnp.pad(W, ((0, _OUT - dout), (0, 0)))
            elif dout < _OUT or din < _OUT:
                W = jnp.pad(W, ((0, _OUT - dout), (0, _OUT - din)))
            ws.append(W)
            bs.append(jnp.pad(B, (0, _OUT - dout)))
    b = jnp.stack(bs)
    return ws, b[:, :, None], b[:, None, :]


def kernel(x, path_lengths, params):
    b, n, d = x.shape
    xf = x.reshape(b * n, d)
    plf = jnp.clip(path_lengths.reshape(b * n), 0, _MAXL - 1)

    onehot = (plf[:, None] == jnp.arange(_MAXL, dtype=jnp.int32)[None, :])
    oh32 = onehot.astype(jnp.int32)
    counts = jnp.sum(oh32, axis=0)
    padded = ((counts + _S - 1) // _S) * _S
    ends = jnp.cumsum(padded)
    starts = ends - padded
    ranks_all = jnp.cumsum(oh32, axis=0) - oh32
    rank = jnp.take_along_axis(ranks_all, plf[:, None], axis=1)[:, 0]
    dest = starts[plf] + rank

    used_supers = ends[-1] // _S
    sbid = jnp.arange(_NSUP, dtype=jnp.int32)
    src_sb = jnp.minimum(sbid, used_supers - 1)
    sel = jnp.searchsorted(ends, src_sb * _S, side="right").astype(jnp.int32)
    sup_used = (sbid < used_supers)

    fbid = jnp.arange(_NBF, dtype=jnp.int32)
    fsel = sel[jnp.minimum(fbid // _G, used_supers - 1)]
    fineu = ((fbid // _G < used_supers)
             & (fbid * _T < starts[fsel] + counts[fsel])).astype(jnp.int32)
    fineu = fineu.reshape(_NSUP, _G)

    ps_tab = jnp.array(_PS_ROWS, dtype=jnp.int32)
    pslot = jnp.where(sup_used[:, None], ps_tab[sel], -1).reshape(-1)
    pidx = jnp.arange(_NP, dtype=jnp.int32)
    lastvalid = jax.lax.cummax(jnp.where(pslot >= 0, pidx, -1))
    psff = pslot[jnp.maximum(lastvalid, 0)].reshape(_NSUP, _NPH)

    valid = pslot >= 0
    vrank = jnp.cumsum(valid.astype(jnp.int32)) - valid.astype(jnp.int32)
    curbuf = (vrank % 2).astype(jnp.int32)
    cand = jnp.where(valid, pidx, _NP + 7)
    sufmin = jax.lax.cummin(cand[::-1])[::-1]
    nxt = jnp.concatenate([sufmin[1:], jnp.array([_NP + 7], jnp.int32)])
    has_next = valid & (nxt < _NP)
    islot = jnp.where(has_next, pslot[jnp.minimum(nxt, _NP - 1)], -1)
    ibuf = jnp.where(has_next, 1 - curbuf, 0).astype(jnp.int32)

    x_sorted = jnp.zeros((_NBF * _T, _IN), dtype=jnp.float32).at[dest].set(xf)

    weights, bc_bank, br_bank = _prep_weights(params)
    depth_b = jnp.array(_DEPTH, dtype=jnp.int32)[sel]
    lvec = jnp.arange(_NPH, dtype=jnp.int32)
    dohid = (sup_used[:, None] & (lvec[None, :] <= depth_b[:, None] - 2)).astype(jnp.int32)
    dofin = (sup_used[:, None] & (lvec[None, :] == _NPH - 1)).astype(jnp.int32)
    y_sorted = _expert_mlp(x_sorted, psff, dohid, dofin, fineu,
                           pslot, curbuf, islot, ibuf,
                           weights, bc_bank, br_bank)

    out = y_sorted[dest]
    return out.reshape(b, n, _OUT)


# speedup vs baseline: 1.7902x; 1.0007x over previous
import jax
import jax.numpy as jnp
from jax.experimental import pallas as pl
from jax.experimental.pallas import tpu as pltpu

_IN = 1024
_OUT = 2048
_MAXL = 8
_T = 256
_G = 4
_S = _T * _G
_NTOK = 8192
_NSUP = _NTOK // _S + _MAXL
_NBF = _NSUP * _G
_NPH = 6
_NP = _NSUP * _NPH
_DEPTH = (3, 3, 4, 4, 5, 5, 5, 5)

_SLOT = []
_sb = 0
for _e in range(_MAXL):
    _SLOT.append([_sb + _j for _j in range(_DEPTH[_e])])
    _sb += _DEPTH[_e]
_NSLOTS = _sb

_DINS = []
for _e in range(_MAXL):
    for _j in range(_DEPTH[_e]):
        _DINS.append(_IN if _j == 0 else _OUT)

_PS_ROWS = []
for _e in range(_MAXL):
    _D = _DEPTH[_e]
    _row = [(_SLOT[_e][_l] if _l <= _D - 2 else -1) for _l in range(_NPH - 1)]
    _row.append(_SLOT[_e][_D - 1])
    _PS_ROWS.append(_row)


def _switch_dma(slot, w_refs, dst_ref, sem_ref, buf, start):
    for i in range(_NSLOTS):
        @pl.when(slot == i)
        def _(i=i):
            cp = pltpu.make_async_copy(
                w_refs[i], dst_ref.at[buf, :, pl.ds(0, _DINS[i])], sem_ref.at[buf])
            if start:
                cp.start()
            else:
                cp.wait()


def _mlp_body(psff_ref, dohid_ref, dofin_ref, fineu_ref,
              wslot_ref, curbuf_ref, islot_ref, ibuf_ref,
              x_ref, bc_ref, br_ref, *rest):
    w_refs = rest[:_NSLOTS]
    o_ref = rest[_NSLOTS]
    wbuf_ref, h_ref, sem_ref = rest[_NSLOTS + 1:]
    sb = pl.program_id(0)
    l = pl.program_id(1)
    g = pl.program_id(2)
    p = sb * _NPH + l

    @pl.when(g == 0)
    def _dma_mgmt():
        @pl.when(p == 0)
        def _bootstrap():
            _switch_dma(wslot_ref[0], w_refs, wbuf_ref, sem_ref,
                        curbuf_ref[0], start=True)

        islot = islot_ref[p]

        @pl.when(islot >= 0)
        def _issue_next():
            _switch_dma(islot, w_refs, wbuf_ref, sem_ref,
                        ibuf_ref[p], start=True)

        wslot = wslot_ref[p]

        @pl.when(wslot >= 0)
        def _wait_cur():
            _switch_dma(wslot, w_refs, wbuf_ref, sem_ref,
                        curbuf_ref[p], start=False)

    fu = fineu_ref[sb, g] == 1
    hid = (dohid_ref[sb, l] == 1) & fu
    fin = (dofin_ref[sb, l] == 1) & fu
    cur = curbuf_ref[p]

    bf = jnp.bfloat16

    @pl.when(hid & (l == 0))
    def _first():
        acc = jax.lax.dot_general(
            wbuf_ref[cur, :, 0:_IN].astype(bf), x_ref[...].astype(bf),
            (((1,), (1,)), ((), ())),
            preferred_element_type=jnp.float32) + bc_ref[0]
        h_ref[g] = jnp.maximum(acc, 0.0)

    @pl.when(hid & (l > 0))
    def _mid():
        acc = jax.lax.dot_general(
            wbuf_ref[cur].astype(bf), h_ref[g].astype(bf),
            (((1,), (0,)), ((), ())),
            preferred_element_type=jnp.float32) + bc_ref[0]
        h_ref[g] = jnp.maximum(acc, 0.0)

    @pl.when(fin)
    def _final():
        o_ref[...] = jax.lax.dot_general(
            h_ref[g].astype(bf), wbuf_ref[cur].astype(bf),
            (((0,), (1,)), ((), ())),
            preferred_element_type=jnp.float32) + br_ref[0]


def _expert_mlp(x_sorted, psff, dohid, dofin, fineu,
                wslot, curbuf, islot, ibuf, weights, bc_bank, br_bank):
    grid_spec = pltpu.PrefetchScalarGridSpec(
        num_scalar_prefetch=8,
        grid=(_NSUP, _NPH, _G),
        in_specs=[
            pl.BlockSpec(
                (_T, _IN),
                lambda sb, l, g, ps, *p: (sb * _G + jnp.where(l == 0, g, _G - 1), 0)),
            pl.BlockSpec(
                (1, _OUT, 1),
                lambda sb, l, g, ps, *p: (ps[sb, l], 0, 0)),
            pl.BlockSpec(
                (1, 1, _OUT),
                lambda sb, l, g, ps, *p: (ps[sb, l], 0, 0)),
        ] + [pl.BlockSpec(memory_space=pltpu.MemorySpace.HBM)] * _NSLOTS,
        out_specs=pl.BlockSpec(
            (_T, _OUT),
            lambda sb, l, g, ps, *p: (sb * _G + jnp.where(l == _NPH - 1, g, 0), 0)),
        scratch_shapes=[
            pltpu.VMEM((2, _OUT, _OUT), jnp.float32),
            pltpu.VMEM((_G, _OUT, _T), jnp.float32),
            pltpu.SemaphoreType.DMA((2,)),
        ],
    )
    return pl.pallas_call(
        _mlp_body,
        grid_spec=grid_spec,
        out_shape=jax.ShapeDtypeStruct((_NBF * _T, _OUT), jnp.float32),
        compiler_params=pltpu.CompilerParams(
            dimension_semantics=("arbitrary", "arbitrary", "arbitrary"),
            fuse_transposed_lhs_in_matmul=True),
    )(psff, dohid, dofin, fineu, wslot, curbuf, islot, ibuf,
      x_sorted, bc_bank, br_bank, *weights)


def _prep_weights(params):
    ws, bs = [], []
    for e in range(_MAXL):
        for j in range(_DEPTH[e]):
            W, B = params[e][j]
            dout, din = W.shape
            if j == 0:
                W = jnp.pad(W, ((0, _OUT - dout), (0, 0)))
            elif dout < _OUT or din < _OUT:
                W = jnp.pad(W, ((0, _OUT - dout), (0, _OUT - din)))
            ws.append(W)
            bs.append(jnp.pad(B, (0, _OUT - dout)))
    b = jnp.stack(bs)
    return ws, b[:, :, None], b[:, None, :]


def kernel(x, path_lengths, params):
    b, n, d = x.shape
    xf = x.reshape(b * n, d)
    plf = jnp.clip(path_lengths.reshape(b * n), 0, _MAXL - 1)

    onehot = (plf[:, None] == jnp.arange(_MAXL, dtype=jnp.int32)[None, :])
    oh32 = onehot.astype(jnp.int32)
    counts = jnp.sum(oh32, axis=0)
    padded = ((counts + _S - 1) // _S) * _S
    ends = jnp.cumsum(padded)
    starts = ends - padded
    ranks_all = jnp.cumsum(oh32, axis=0) - oh32
    rank = jnp.take_along_axis(ranks_all, plf[:, None], axis=1)[:, 0]
    dest = starts[plf] + rank

    used_supers = ends[-1] // _S
    sbid = jnp.arange(_NSUP, dtype=jnp.int32)
    src_sb = jnp.minimum(sbid, used_supers - 1)
    sel = jnp.searchsorted(ends, src_sb * _S, side="right").astype(jnp.int32)
    sup_used = (sbid < used_supers)

    fbid = jnp.arange(_NBF, dtype=jnp.int32)
    fsel = sel[jnp.minimum(fbid // _G, used_supers - 1)]
    fineu = ((fbid // _G < used_supers)
             & (fbid * _T < starts[fsel] + counts[fsel])).astype(jnp.int32)
    fineu = fineu.reshape(_NSUP, _G)

    ps_tab = jnp.array(_PS_ROWS, dtype=jnp.int32)
    pslot = jnp.where(sup_used[:, None], ps_tab[sel], -1).reshape(-1)
    pidx = jnp.arange(_NP, dtype=jnp.int32)
    lastvalid = jax.lax.cummax(jnp.where(pslot >= 0, pidx, -1))
    psff = pslot[jnp.maximum(lastvalid, 0)].reshape(_NSUP, _NPH)

    valid = pslot >= 0
    vrank = jnp.cumsum(valid.astype(jnp.int32)) - valid.astype(jnp.int32)
    curbuf = (vrank % 2).astype(jnp.int32)
    cand = jnp.where(valid, pidx, _NP + 7)
    sufmin = jax.lax.cummin(cand[::-1])[::-1]
    nxt = jnp.concatenate([sufmin[1:], jnp.array([_NP + 7], jnp.int32)])
    has_next = valid & (nxt < _NP)
    islot = jnp.where(has_next, pslot[jnp.minimum(nxt, _NP - 1)], -1)
    ibuf = jnp.where(has_next, 1 - curbuf, 0).astype(jnp.int32)

    x_sorted = jnp.zeros((_NBF * _T, _IN), dtype=jnp.float32).at[dest].set(xf)

    weights, bc_bank, br_bank = _prep_weights(params)
    depth_b = jnp.array(_DEPTH, dtype=jnp.int32)[sel]
    lvec = jnp.arange(_NPH, dtype=jnp.int32)
    dohid = (sup_used[:, None] & (lvec[None, :] <= depth_b[:, None] - 2)).astype(jnp.int32)
    dofin = (sup_used[:, None] & (lvec[None, :] == _NPH - 1)).astype(jnp.int32)
    y_sorted = _expert_mlp(x_sorted, psff, dohid, dofin, fineu,
                           pslot, curbuf, islot, ibuf,
                           weights, bc_bank, br_bank)

    out = y_sorted[dest]
    return out.reshape(b, n, _OUT)


# speedup vs baseline: 1.8637x; 1.0411x over previous
import jax
import jax.numpy as jnp
from jax.experimental import pallas as pl
from jax.experimental.pallas import tpu as pltpu

_IN = 1024
_OUT = 2048
_MAXL = 8
_T = 512
_G = 2
_S = _T * _G
_NTOK = 8192
_NSUP = _NTOK // _S + _MAXL
_NBF = _NSUP * _G
_NPH = 6
_NP = _NSUP * _NPH
_DEPTH = (3, 3, 4, 4, 5, 5, 5, 5)

_SLOT = []
_sb = 0
for _e in range(_MAXL):
    _SLOT.append([_sb + _j for _j in range(_DEPTH[_e])])
    _sb += _DEPTH[_e]
_NSLOTS = _sb

_DINS = []
for _e in range(_MAXL):
    for _j in range(_DEPTH[_e]):
        _DINS.append(_IN if _j == 0 else _OUT)

_PS_ROWS = []
for _e in range(_MAXL):
    _D = _DEPTH[_e]
    _row = [(_SLOT[_e][_l] if _l <= _D - 2 else -1) for _l in range(_NPH - 1)]
    _row.append(_SLOT[_e][_D - 1])
    _PS_ROWS.append(_row)


def _switch_dma(slot, w_refs, dst_ref, sem_ref, buf, start):
    for i in range(_NSLOTS):
        @pl.when(slot == i)
        def _(i=i):
            cp = pltpu.make_async_copy(
                w_refs[i], dst_ref.at[buf, :, pl.ds(0, _DINS[i])], sem_ref.at[buf])
            if start:
                cp.start()
            else:
                cp.wait()


def _mlp_body(psff_ref, dohid_ref, dofin_ref, fineu_ref,
              wslot_ref, curbuf_ref, islot_ref, ibuf_ref,
              x_ref, bc_ref, br_ref, *rest):
    w_refs = rest[:_NSLOTS]
    o_ref = rest[_NSLOTS]
    wbuf_ref, h_ref, sem_ref = rest[_NSLOTS + 1:]
    sb = pl.program_id(0)
    l = pl.program_id(1)
    g = pl.program_id(2)
    p = sb * _NPH + l

    @pl.when(g == 0)
    def _dma_mgmt():
        @pl.when(p == 0)
        def _bootstrap():
            _switch_dma(wslot_ref[0], w_refs, wbuf_ref, sem_ref,
                        curbuf_ref[0], start=True)

        islot = islot_ref[p]

        @pl.when(islot >= 0)
        def _issue_next():
            _switch_dma(islot, w_refs, wbuf_ref, sem_ref,
                        ibuf_ref[p], start=True)

        wslot = wslot_ref[p]

        @pl.when(wslot >= 0)
        def _wait_cur():
            _switch_dma(wslot, w_refs, wbuf_ref, sem_ref,
                        curbuf_ref[p], start=False)

    fu = fineu_ref[sb, g] == 1
    hid = (dohid_ref[sb, l] == 1) & fu
    fin = (dofin_ref[sb, l] == 1) & fu
    cur = curbuf_ref[p]

    @pl.when(hid & (l == 0))
    def _first():
        acc = jax.lax.dot_general(
            wbuf_ref[cur, :, 0:_IN], x_ref[...], (((1,), (1,)), ((), ())),
            preferred_element_type=jnp.float32) + bc_ref[0]
        h_ref[g] = jnp.maximum(acc, 0.0)

    @pl.when(hid & (l > 0))
    def _mid():
        acc = jax.lax.dot_general(
            wbuf_ref[cur], h_ref[g], (((1,), (0,)), ((), ())),
            preferred_element_type=jnp.float32) + bc_ref[0]
        h_ref[g] = jnp.maximum(acc, 0.0)

    @pl.when(fin)
    def _final():
        o_ref[...] = jax.lax.dot_general(
            h_ref[g], wbuf_ref[cur], (((0,), (1,)), ((), ())),
            preferred_element_type=jnp.float32) + br_ref[0]


def _expert_mlp(x_sorted, psff, dohid, dofin, fineu,
                wslot, curbuf, islot, ibuf, weights, bc_bank, br_bank):
    grid_spec = pltpu.PrefetchScalarGridSpec(
        num_scalar_prefetch=8,
        grid=(_NSUP, _NPH, _G),
        in_specs=[
            pl.BlockSpec(
                (_T, _IN),
                lambda sb, l, g, ps, *p: (sb * _G + jnp.where(l == 0, g, _G - 1), 0)),
            pl.BlockSpec(
                (1, _OUT, 1),
                lambda sb, l, g, ps, *p: (ps[sb, l], 0, 0)),
            pl.BlockSpec(
                (1, 1, _OUT),
                lambda sb, l, g, ps, *p: (ps[sb, l], 0, 0)),
        ] + [pl.BlockSpec(memory_space=pltpu.MemorySpace.HBM)] * _NSLOTS,
        out_specs=pl.BlockSpec(
            (_T, _OUT),
            lambda sb, l, g, ps, *p: (sb * _G + jnp.where(l == _NPH - 1, g, 0), 0)),
        scratch_shapes=[
            pltpu.VMEM((2, _OUT, _OUT), jnp.float32),
            pltpu.VMEM((_G, _OUT, _T), jnp.float32),
            pltpu.SemaphoreType.DMA((2,)),
        ],
    )
    return pl.pallas_call(
        _mlp_body,
        grid_spec=grid_spec,
        out_shape=jax.ShapeDtypeStruct((_NBF * _T, _OUT), jnp.float32),
        compiler_params=pltpu.CompilerParams(
            dimension_semantics=("arbitrary", "arbitrary", "arbitrary"),
            fuse_transposed_lhs_in_matmul=True),
    )(psff, dohid, dofin, fineu, wslot, curbuf, islot, ibuf,
      x_sorted, bc_bank, br_bank, *weights)


def _prep_weights(params):
    ws, bs = [], []
    for e in range(_MAXL):
        for j in range(_DEPTH[e]):
            W, B = params[e][j]
            dout, din = W.shape
            if j == 0:
                W = jnp.pad(W, ((0, _OUT - dout), (0, 0)))
            elif dout < _OUT or din < _OUT:
                W = jnp.pad(W, ((0, _OUT - dout), (0, _OUT - din)))
            ws.append(W)
            bs.append(jnp.pad(B, (0, _OUT - dout)))
    b = jnp.stack(bs)
    return ws, b[:, :, None], b[:, None, :]


def kernel(x, path_lengths, params):
    b, n, d = x.shape
    xf = x.reshape(b * n, d)
    plf = jnp.clip(path_lengths.reshape(b * n), 0, _MAXL - 1)

    onehot = (plf[:, None] == jnp.arange(_MAXL, dtype=jnp.int32)[None, :])
    oh32 = onehot.astype(jnp.int32)
    counts = jnp.sum(oh32, axis=0)
    padded = ((counts + _S - 1) // _S) * _S
    ends = jnp.cumsum(padded)
    starts = ends - padded
    ranks_all = jnp.cumsum(oh32, axis=0) - oh32
    rank = jnp.take_along_axis(ranks_all, plf[:, None], axis=1)[:, 0]
    dest = starts[plf] + rank

    used_supers = ends[-1] // _S
    sbid = jnp.arange(_NSUP, dtype=jnp.int32)
    src_sb = jnp.minimum(sbid, used_supers - 1)
    sel = jnp.searchsorted(ends, src_sb * _S, side="right").astype(jnp.int32)
    sup_used = (sbid < used_supers)

    fbid = jnp.arange(_NBF, dtype=jnp.int32)
    fsel = sel[jnp.minimum(fbid // _G, used_supers - 1)]
    fineu = ((fbid // _G < used_supers)
             & (fbid * _T < starts[fsel] + counts[fsel])).astype(jnp.int32)
    fineu = fineu.reshape(_NSUP, _G)

    ps_tab = jnp.array(_PS_ROWS, dtype=jnp.int32)
    pslot = jnp.where(sup_used[:, None], ps_tab[sel], -1).reshape(-1)
    pidx = jnp.arange(_NP, dtype=jnp.int32)
    lastvalid = jax.lax.cummax(jnp.where(pslot >= 0, pidx, -1))
    psff = pslot[jnp.maximum(lastvalid, 0)].reshape(_NSUP, _NPH)

    valid = pslot >= 0
    vrank = jnp.cumsum(valid.astype(jnp.int32)) - valid.astype(jnp.int32)
    curbuf = (vrank % 2).astype(jnp.int32)
    cand = jnp.where(valid, pidx, _NP + 7)
    sufmin = jax.lax.cummin(cand[::-1])[::-1]
    nxt = jnp.concatenate([sufmin[1:], jnp.array([_NP + 7], jnp.int32)])
    has_next = valid & (nxt < _NP)
    islot = jnp.where(has_next, pslot[jnp.minimum(nxt, _NP - 1)], -1)
    ibuf = jnp.where(has_next, 1 - curbuf, 0).astype(jnp.int32)

    x_sorted = jnp.zeros((_NBF * _T, _IN), dtype=jnp.float32).at[dest].set(xf)

    weights, bc_bank, br_bank = _prep_weights(params)
    depth_b = jnp.array(_DEPTH, dtype=jnp.int32)[sel]
    lvec = jnp.arange(_NPH, dtype=jnp.int32)
    dohid = (sup_used[:, None] & (lvec[None, :] <= depth_b[:, None] - 2)).astype(jnp.int32)
    dofin = (sup_used[:, None] & (lvec[None, :] == _NPH - 1)).astype(jnp.int32)
    y_sorted = _expert_mlp(x_sorted, psff, dohid, dofin, fineu,
                           pslot, curbuf, islot, ibuf,
                           weights, bc_bank, br_bank)

    out = y_sorted[dest]
    return out.reshape(b, n, _OUT)


# speedup vs baseline: 2.2009x; 1.1809x over previous
import functools

import jax
import jax.numpy as jnp
from jax import lax
from jax.experimental import pallas as pl
from jax.experimental.pallas import tpu as pltpu
from jax.experimental.pallas import tpu_sc as plsc

_IN = 1024
_OUT = 2048
_MAXL = 8
_T = 512
_G = 2
_S = _T * _G
_NTOK = 8192
_NSUP = _NTOK // _S + _MAXL
_NBF = _NSUP * _G
_NPH = 6
_NP = _NSUP * _NPH
_DEPTH = (3, 3, 4, 4, 5, 5, 5, 5)

_SLOT = []
_sb = 0
for _e in range(_MAXL):
    _SLOT.append([_sb + _j for _j in range(_DEPTH[_e])])
    _sb += _DEPTH[_e]
_NSLOTS = _sb

_DINS = []
for _e in range(_MAXL):
    for _j in range(_DEPTH[_e]):
        _DINS.append(_IN if _j == 0 else _OUT)

_PS_ROWS = []
for _e in range(_MAXL):
    _D = _DEPTH[_e]
    _row = [(_SLOT[_e][_l] if _l <= _D - 2 else -1) for _l in range(_NPH - 1)]
    _row.append(_SLOT[_e][_D - 1])
    _PS_ROWS.append(_row)


def _switch_dma(slot, w_refs, dst_ref, sem_ref, buf, start):
    for i in range(_NSLOTS):
        @pl.when(slot == i)
        def _(i=i):
            cp = pltpu.make_async_copy(
                w_refs[i], dst_ref.at[buf, :, pl.ds(0, _DINS[i])], sem_ref.at[buf])
            if start:
                cp.start()
            else:
                cp.wait()


def _mlp_body(psff_ref, dohid_ref, dofin_ref, fineu_ref,
              wslot_ref, curbuf_ref, islot_ref, ibuf_ref,
              x_ref, bc_ref, br_ref, *rest):
    w_refs = rest[:_NSLOTS]
    o_ref = rest[_NSLOTS]
    wbuf_ref, h_ref, sem_ref = rest[_NSLOTS + 1:]
    sb = pl.program_id(0)
    l = pl.program_id(1)
    g = pl.program_id(2)
    p = sb * _NPH + l

    @pl.when(g == 0)
    def _dma_mgmt():
        @pl.when(p == 0)
        def _bootstrap():
            _switch_dma(wslot_ref[0], w_refs, wbuf_ref, sem_ref,
                        curbuf_ref[0], start=True)

        islot = islot_ref[p]

        @pl.when(islot >= 0)
        def _issue_next():
            _switch_dma(islot, w_refs, wbuf_ref, sem_ref,
                        ibuf_ref[p], start=True)

        wslot = wslot_ref[p]

        @pl.when(wslot >= 0)
        def _wait_cur():
            _switch_dma(wslot, w_refs, wbuf_ref, sem_ref,
                        curbuf_ref[p], start=False)

    fu = fineu_ref[sb, g] == 1
    hid = (dohid_ref[sb, l] == 1) & fu
    fin = (dofin_ref[sb, l] == 1) & fu
    cur = curbuf_ref[p]

    @pl.when(hid & (l == 0))
    def _first():
        acc = jax.lax.dot_general(
            wbuf_ref[cur, :, 0:_IN], x_ref[...], (((1,), (1,)), ((), ())),
            preferred_element_type=jnp.float32) + bc_ref[0]
        h_ref[g] = jnp.maximum(acc, 0.0)

    @pl.when(hid & (l > 0))
    def _mid():
        acc = jax.lax.dot_general(
            wbuf_ref[cur], h_ref[g], (((1,), (0,)), ((), ())),
            preferred_element_type=jnp.float32) + bc_ref[0]
        h_ref[g] = jnp.maximum(acc, 0.0)

    @pl.when(fin)
    def _final():
        o_ref[...] = jax.lax.dot_general(
            h_ref[g], wbuf_ref[cur], (((0,), (1,)), ((), ())),
            preferred_element_type=jnp.float32) + br_ref[0]


def _expert_mlp(x_sorted, psff, dohid, dofin, fineu,
                wslot, curbuf, islot, ibuf, weights, bc_bank, br_bank):
    grid_spec = pltpu.PrefetchScalarGridSpec(
        num_scalar_prefetch=8,
        grid=(_NSUP, _NPH, _G),
        in_specs=[
            pl.BlockSpec(
                (_T, _IN),
                lambda sb, l, g, ps, *p: (sb * _G + jnp.where(l == 0, g, _G - 1), 0)),
            pl.BlockSpec(
                (1, _OUT, 1),
                lambda sb, l, g, ps, *p: (ps[sb, l], 0, 0)),
            pl.BlockSpec(
                (1, 1, _OUT),
                lambda sb, l, g, ps, *p: (ps[sb, l], 0, 0)),
        ] + [pl.BlockSpec(memory_space=pltpu.MemorySpace.HBM)] * _NSLOTS,
        out_specs=pl.BlockSpec(
            (_T, _OUT),
            lambda sb, l, g, ps, *p: (sb * _G + jnp.where(l == _NPH - 1, g, 0), 0)),
        scratch_shapes=[
            pltpu.VMEM((2, _OUT, _OUT), jnp.float32),
            pltpu.VMEM((_G, _OUT, _T), jnp.float32),
            pltpu.SemaphoreType.DMA((2,)),
        ],
    )
    return pl.pallas_call(
        _mlp_body,
        grid_spec=grid_spec,
        out_shape=jax.ShapeDtypeStruct((_NBF * _T, _OUT), jnp.float32),
        compiler_params=pltpu.CompilerParams(
            dimension_semantics=("arbitrary", "arbitrary", "arbitrary"),
            fuse_transposed_lhs_in_matmul=True),
    )(psff, dohid, dofin, fineu, wslot, curbuf, islot, ibuf,
      x_sorted, bc_bank, br_bank, *weights)


def _prep_weights(params):
    ws, bs = [], []
    for e in range(_MAXL):
        for j in range(_DEPTH[e]):
            W, B = params[e][j]
            dout, din = W.shape
            if j == 0:
                W = jnp.pad(W, ((0, _OUT - dout), (0, 0)))
            elif dout < _OUT or din < _OUT:
                W = jnp.pad(W, ((0, _OUT - dout), (0, _OUT - din)))
            ws.append(W)
            bs.append(jnp.pad(B, (0, _OUT - dout)))
    b = jnp.stack(bs)
    return ws, b[:, :, None], b[:, None, :]


_NW = 32
_BPW = _NTOK // _NW
_CH = 16


def _sc_dispatch(xf, dest):
    mesh = plsc.VectorSubcoreMesh(core_axis_name="c", subcore_axis_name="s")

    @functools.partial(
        pl.kernel, mesh=mesh,
        out_type=jax.ShapeDtypeStruct((_NBF * _T, _IN), jnp.float32),
        scratch_types=[pltpu.VMEM((_CH,), jnp.int32),
                       pltpu.VMEM((_CH, _IN), jnp.float32)],
    )
    def k(xf_hbm, dest_hbm, out_hbm, idx_v, rows_v):
        wid = lax.axis_index("s") * 2 + lax.axis_index("c")
        base = wid * _BPW
        for c in range(_BPW // _CH):
            off = base + c * _CH
            pltpu.sync_copy(dest_hbm.at[pl.ds(off, _CH)], idx_v)
            pltpu.sync_copy(xf_hbm.at[pl.ds(off, _CH)], rows_v)
            pltpu.sync_copy(rows_v, out_hbm.at[idx_v])

    return k(xf, dest)


def _sc_combine(y_sorted, dest):
    mesh = plsc.VectorSubcoreMesh(core_axis_name="c", subcore_axis_name="s")

    @functools.partial(
        pl.kernel, mesh=mesh,
        out_type=jax.ShapeDtypeStruct((_NTOK, _OUT), jnp.float32),
        scratch_types=[pltpu.VMEM((_CH,), jnp.int32),
                       pltpu.VMEM((_CH, _OUT), jnp.float32)],
    )
    def k(y_hbm, dest_hbm, out_hbm, idx_v, rows_v):
        wid = lax.axis_index("s") * 2 + lax.axis_index("c")
        base = wid * _BPW
        for c in range(_BPW // _CH):
            off = base + c * _CH
            pltpu.sync_copy(dest_hbm.at[pl.ds(off, _CH)], idx_v)
            pltpu.sync_copy(y_hbm.at[idx_v], rows_v)
            pltpu.sync_copy(rows_v, out_hbm.at[pl.ds(off, _CH)])

    return k(y_sorted, dest)


def kernel(x, path_lengths, params):
    b, n, d = x.shape
    xf = x.reshape(b * n, d)
    plf = jnp.clip(path_lengths.reshape(b * n), 0, _MAXL - 1)

    onehot = (plf[:, None] == jnp.arange(_MAXL, dtype=jnp.int32)[None, :])
    oh32 = onehot.astype(jnp.int32)
    counts = jnp.sum(oh32, axis=0)
    padded = ((counts + _S - 1) // _S) * _S
    ends = jnp.cumsum(padded)
    starts = ends - padded
    ranks_all = jnp.cumsum(oh32, axis=0) - oh32
    rank = jnp.take_along_axis(ranks_all, plf[:, None], axis=1)[:, 0]
    dest = starts[plf] + rank

    used_supers = ends[-1] // _S
    sbid = jnp.arange(_NSUP, dtype=jnp.int32)
    src_sb = jnp.minimum(sbid, used_supers - 1)
    sel = jnp.searchsorted(ends, src_sb * _S, side="right").astype(jnp.int32)
    sup_used = (sbid < used_supers)

    fbid = jnp.arange(_NBF, dtype=jnp.int32)
    fsel = sel[jnp.minimum(fbid // _G, used_supers - 1)]
    fineu = ((fbid // _G < used_supers)
             & (fbid * _T < starts[fsel] + counts[fsel])).astype(jnp.int32)
    fineu = fineu.reshape(_NSUP, _G)

    ps_tab = jnp.array(_PS_ROWS, dtype=jnp.int32)
    pslot = jnp.where(sup_used[:, None], ps_tab[sel], -1).reshape(-1)
    pidx = jnp.arange(_NP, dtype=jnp.int32)
    lastvalid = jax.lax.cummax(jnp.where(pslot >= 0, pidx, -1))
    psff = pslot[jnp.maximum(lastvalid, 0)].reshape(_NSUP, _NPH)

    valid = pslot >= 0
    vrank = jnp.cumsum(valid.astype(jnp.int32)) - valid.astype(jnp.int32)
    curbuf = (vrank % 2).astype(jnp.int32)
    cand = jnp.where(valid, pidx, _NP + 7)
    sufmin = jax.lax.cummin(cand[::-1])[::-1]
    nxt = jnp.concatenate([sufmin[1:], jnp.array([_NP + 7], jnp.int32)])
    has_next = valid & (nxt < _NP)
    islot = jnp.where(has_next, pslot[jnp.minimum(nxt, _NP - 1)], -1)
    ibuf = jnp.where(has_next, 1 - curbuf, 0).astype(jnp.int32)

    x_sorted = _sc_dispatch(xf, dest)

    weights, bc_bank, br_bank = _prep_weights(params)
    depth_b = jnp.array(_DEPTH, dtype=jnp.int32)[sel]
    lvec = jnp.arange(_NPH, dtype=jnp.int32)
    dohid = (sup_used[:, None] & (lvec[None, :] <= depth_b[:, None] - 2)).astype(jnp.int32)
    dofin = (sup_used[:, None] & (lvec[None, :] == _NPH - 1)).astype(jnp.int32)
    y_sorted = _expert_mlp(x_sorted, psff, dohid, dofin, fineu,
                           pslot, curbuf, islot, ibuf,
                           weights, bc_bank, br_bank)

    out = _sc_combine(y_sorted, dest)
    return out.reshape(b, n, _OUT)


# speedup vs baseline: 2.2494x; 1.0220x over previous
import functools

import jax
import jax.numpy as jnp
from jax import lax
from jax.experimental import pallas as pl
from jax.experimental.pallas import tpu as pltpu
from jax.experimental.pallas import tpu_sc as plsc

_IN = 1024
_OUT = 2048
_MAXL = 8
_T = 512
_G = 2
_S = _T * _G
_NTOK = 8192
_NSUP = _NTOK // _S + _MAXL
_NBF = _NSUP * _G
_NPH = 6
_NP = _NSUP * _NPH
_DEPTH = (3, 3, 4, 4, 5, 5, 5, 5)

_SLOT = []
_sb = 0
for _e in range(_MAXL):
    _SLOT.append([_sb + _j for _j in range(_DEPTH[_e])])
    _sb += _DEPTH[_e]
_NSLOTS = _sb

_DINS = []
for _e in range(_MAXL):
    for _j in range(_DEPTH[_e]):
        _DINS.append(_IN if _j == 0 else _OUT)

_PS_ROWS = []
for _e in range(_MAXL):
    _D = _DEPTH[_e]
    _row = [(_SLOT[_e][_l] if _l <= _D - 2 else -1) for _l in range(_NPH - 1)]
    _row.append(_SLOT[_e][_D - 1])
    _PS_ROWS.append(_row)


def _switch_dma(slot, w_refs, dst_ref, sem_ref, buf, start):
    for i in range(_NSLOTS):
        @pl.when(slot == i)
        def _(i=i):
            cp = pltpu.make_async_copy(
                w_refs[i], dst_ref.at[buf, :, pl.ds(0, _DINS[i])], sem_ref.at[buf])
            if start:
                cp.start()
            else:
                cp.wait()


def _mlp_body(psff_ref, dohid_ref, dofin_ref, fineu_ref,
              wslot_ref, curbuf_ref, islot_ref, ibuf_ref,
              x_ref, bc_ref, br_ref, *rest):
    w_refs = rest[:_NSLOTS]
    o_ref = rest[_NSLOTS]
    wbuf_ref, h_ref, sem_ref = rest[_NSLOTS + 1:]
    sb = pl.program_id(0)
    l = pl.program_id(1)
    g = pl.program_id(2)
    p = sb * _NPH + l

    @pl.when(g == 0)
    def _dma_mgmt():
        @pl.when(p == 0)
        def _bootstrap():
            _switch_dma(wslot_ref[0], w_refs, wbuf_ref, sem_ref,
                        curbuf_ref[0], start=True)

        islot = islot_ref[p]

        @pl.when(islot >= 0)
        def _issue_next():
            _switch_dma(islot, w_refs, wbuf_ref, sem_ref,
                        ibuf_ref[p], start=True)

        wslot = wslot_ref[p]

        @pl.when(wslot >= 0)
        def _wait_cur():
            _switch_dma(wslot, w_refs, wbuf_ref, sem_ref,
                        curbuf_ref[p], start=False)

    fu = fineu_ref[sb, g] == 1
    hid = (dohid_ref[sb, l] == 1) & fu
    fin = (dofin_ref[sb, l] == 1) & fu
    cur = curbuf_ref[p]

    @pl.when(hid & (l == 0))
    def _first():
        acc = jax.lax.dot_general(
            wbuf_ref[cur, :, 0:_IN], x_ref[...], (((1,), (1,)), ((), ())),
            preferred_element_type=jnp.float32) + bc_ref[0]
        h_ref[g] = jnp.maximum(acc, 0.0)

    @pl.when(hid & (l > 0))
    def _mid():
        acc = jax.lax.dot_general(
            wbuf_ref[cur], h_ref[g], (((1,), (0,)), ((), ())),
            preferred_element_type=jnp.float32) + bc_ref[0]
        h_ref[g] = jnp.maximum(acc, 0.0)

    @pl.when(fin)
    def _final():
        o_ref[...] = jax.lax.dot_general(
            h_ref[g], wbuf_ref[cur], (((0,), (1,)), ((), ())),
            preferred_element_type=jnp.float32) + br_ref[0]


def _expert_mlp(x_sorted, psff, dohid, dofin, fineu,
                wslot, curbuf, islot, ibuf, weights, bc_bank, br_bank):
    grid_spec = pltpu.PrefetchScalarGridSpec(
        num_scalar_prefetch=8,
        grid=(_NSUP, _NPH, _G),
        in_specs=[
            pl.BlockSpec(
                (_T, _IN),
                lambda sb, l, g, ps, *p: (sb * _G + jnp.where(l == 0, g, _G - 1), 0)),
            pl.BlockSpec(
                (1, _OUT, 1),
                lambda sb, l, g, ps, *p: (ps[sb, l], 0, 0)),
            pl.BlockSpec(
                (1, 1, _OUT),
                lambda sb, l, g, ps, *p: (ps[sb, l], 0, 0)),
        ] + [pl.BlockSpec(memory_space=pltpu.MemorySpace.HBM)] * _NSLOTS,
        out_specs=pl.BlockSpec(
            (_T, _OUT),
            lambda sb, l, g, ps, *p: (sb * _G + jnp.where(l == _NPH - 1, g, 0), 0)),
        scratch_shapes=[
            pltpu.VMEM((2, _OUT, _OUT), jnp.float32),
            pltpu.VMEM((_G, _OUT, _T), jnp.float32),
            pltpu.SemaphoreType.DMA((2,)),
        ],
    )
    return pl.pallas_call(
        _mlp_body,
        grid_spec=grid_spec,
        out_shape=jax.ShapeDtypeStruct((_NBF * _T, _OUT), jnp.float32),
        compiler_params=pltpu.CompilerParams(
            dimension_semantics=("arbitrary", "arbitrary", "arbitrary"),
            fuse_transposed_lhs_in_matmul=True),
    )(psff, dohid, dofin, fineu, wslot, curbuf, islot, ibuf,
      x_sorted, bc_bank, br_bank, *weights)


def _prep_weights(params):
    ws, bs = [], []
    for e in range(_MAXL):
        for j in range(_DEPTH[e]):
            W, B = params[e][j]
            dout, din = W.shape
            if j == 0:
                W = jnp.pad(W, ((0, _OUT - dout), (0, 0)))
            elif dout < _OUT or din < _OUT:
                W = jnp.pad(W, ((0, _OUT - dout), (0, _OUT - din)))
            ws.append(W)
            bs.append(jnp.pad(B, (0, _OUT - dout)))
    b = jnp.stack(bs)
    return ws, b[:, :, None], b[:, None, :]


_NW = 32
_BPW = _NTOK // _NW
_CH = 32


def _sc_dispatch(xf, dest):
    mesh = plsc.VectorSubcoreMesh(core_axis_name="c", subcore_axis_name="s")

    @functools.partial(
        pl.kernel, mesh=mesh,
        out_type=jax.ShapeDtypeStruct((_NBF * _T, _IN), jnp.float32),
        scratch_types=[pltpu.VMEM((_CH,), jnp.int32),
                       pltpu.VMEM((_CH, _IN), jnp.float32)],
    )
    def k(xf_hbm, dest_hbm, out_hbm, idx_v, rows_v):
        wid = lax.axis_index("s") * 2 + lax.axis_index("c")
        base = wid * _BPW
        for c in range(_BPW // _CH):
            off = base + c * _CH
            pltpu.sync_copy(dest_hbm.at[pl.ds(off, _CH)], idx_v)
            pltpu.sync_copy(xf_hbm.at[pl.ds(off, _CH)], rows_v)
            pltpu.sync_copy(rows_v, out_hbm.at[idx_v])

    return k(xf, dest)


def _sc_combine(y_sorted, dest):
    mesh = plsc.VectorSubcoreMesh(core_axis_name="c", subcore_axis_name="s")

    @functools.partial(
        pl.kernel, mesh=mesh,
        out_type=jax.ShapeDtypeStruct((_NTOK, _OUT), jnp.float32),
        scratch_types=[pltpu.VMEM((_CH,), jnp.int32),
                       pltpu.VMEM((_CH, _OUT), jnp.float32)],
    )
    def k(y_hbm, dest_hbm, out_hbm, idx_v, rows_v):
        wid = lax.axis_index("s") * 2 + lax.axis_index("c")
        base = wid * _BPW
        for c in range(_BPW // _CH):
            off = base + c * _CH
            pltpu.sync_copy(dest_hbm.at[pl.ds(off, _CH)], idx_v)
            pltpu.sync_copy(y_hbm.at[idx_v], rows_v)
            pltpu.sync_copy(rows_v, out_hbm.at[pl.ds(off, _CH)])

    return k(y_sorted, dest)


def kernel(x, path_lengths, params):
    b, n, d = x.shape
    xf = x.reshape(b * n, d)
    plf = jnp.clip(path_lengths.reshape(b * n), 0, _MAXL - 1)

    onehot = (plf[:, None] == jnp.arange(_MAXL, dtype=jnp.int32)[None, :])
    oh32 = onehot.astype(jnp.int32)
    counts = jnp.sum(oh32, axis=0)
    padded = ((counts + _S - 1) // _S) * _S
    ends = jnp.cumsum(padded)
    starts = ends - padded
    ranks_all = jnp.cumsum(oh32, axis=0) - oh32
    rank = jnp.take_along_axis(ranks_all, plf[:, None], axis=1)[:, 0]
    dest = starts[plf] + rank

    used_supers = ends[-1] // _S
    sbid = jnp.arange(_NSUP, dtype=jnp.int32)
    src_sb = jnp.minimum(sbid, used_supers - 1)
    sel = jnp.searchsorted(ends, src_sb * _S, side="right").astype(jnp.int32)
    sup_used = (sbid < used_supers)

    fbid = jnp.arange(_NBF, dtype=jnp.int32)
    fsel = sel[jnp.minimum(fbid // _G, used_supers - 1)]
    fineu = ((fbid // _G < used_supers)
             & (fbid * _T < starts[fsel] + counts[fsel])).astype(jnp.int32)
    fineu = fineu.reshape(_NSUP, _G)

    ps_tab = jnp.array(_PS_ROWS, dtype=jnp.int32)
    pslot = jnp.where(sup_used[:, None], ps_tab[sel], -1).reshape(-1)
    pidx = jnp.arange(_NP, dtype=jnp.int32)
    lastvalid = jax.lax.cummax(jnp.where(pslot >= 0, pidx, -1))
    psff = pslot[jnp.maximum(lastvalid, 0)].reshape(_NSUP, _NPH)

    valid = pslot >= 0
    vrank = jnp.cumsum(valid.astype(jnp.int32)) - valid.astype(jnp.int32)
    curbuf = (vrank % 2).astype(jnp.int32)
    cand = jnp.where(valid, pidx, _NP + 7)
    sufmin = jax.lax.cummin(cand[::-1])[::-1]
    nxt = jnp.concatenate([sufmin[1:], jnp.array([_NP + 7], jnp.int32)])
    has_next = valid & (nxt < _NP)
    islot = jnp.where(has_next, pslot[jnp.minimum(nxt, _NP - 1)], -1)
    ibuf = jnp.where(has_next, 1 - curbuf, 0).astype(jnp.int32)

    x_sorted = _sc_dispatch(xf, dest)

    weights, bc_bank, br_bank = _prep_weights(params)
    depth_b = jnp.array(_DEPTH, dtype=jnp.int32)[sel]
    lvec = jnp.arange(_NPH, dtype=jnp.int32)
    dohid = (sup_used[:, None] & (lvec[None, :] <= depth_b[:, None] - 2)).astype(jnp.int32)
    dofin = (sup_used[:, None] & (lvec[None, :] == _NPH - 1)).astype(jnp.int32)
    y_sorted = _expert_mlp(x_sorted, psff, dohid, dofin, fineu,
                           pslot, curbuf, islot, ibuf,
                           weights, bc_bank, br_bank)

    out = _sc_combine(y_sorted, dest)
    return out.reshape(b, n, _OUT)
